# baseline (speedup 1.0000x reference)
import jax
import jax.numpy as jnp
from jax import lax
from jax.experimental import pallas as pl
from jax.experimental.pallas import tpu as pltpu
from jax.experimental.pallas import tpu_sc as plsc

N = 10000
E = 320000
D = 128
NC = 2
NS = 16
NPAD = 10240
CHUNK = 128
EPAD = 327680
EPT_A = EPAD // NS
EPT_C = EPAD // (NC * NS)
A_CHUNKS = EPT_A // CHUNK
C_CHUNKS = EPT_C // CHUNK
ROWS_PER_TILE = NPAD // NS
WB = ROWS_PER_TILE // CHUNK
NBUF = 2
GRP = C_CHUNKS // 2
CH = 64
DW = D // 2


def _rsqrt16(d):
    i = lax.bitcast_convert_type(d, jnp.int32)
    magic = jnp.full((16,), 0x5F3759DF, jnp.int32)
    y = lax.bitcast_convert_type(magic - lax.shift_right_logical(i, 1),
                                 jnp.float32)
    for _ in range(3):
        y = y * (1.5 - 0.5 * d * y * y)
    return y


def _deg_body(dst_hbm, dinv_hbm, dst_v, ones_v, buf_v, deg_sh, dsem):
    cid = lax.axis_index("c")
    sid = lax.axis_index("s")

    @pl.when(cid == 0)
    def _():
        one16 = jnp.ones((16,), jnp.float32)
        for i in range(CHUNK // 16):
            ones_v[pl.ds(i * 16, 16)] = one16
        for i in range(ROWS_PER_TILE // 16):
            buf_v[pl.ds(i * 16, 16)] = one16
        base = sid * ROWS_PER_TILE
        pltpu.sync_copy(buf_v, deg_sh.at[pl.ds(base, ROWS_PER_TILE)])
        plsc.subcore_barrier()
        pltpu.sync_copy(dst_hbm.at[sid], dst_v)

        def hist_step(j, c):
            b = lax.rem(j, NBUF)

            @pl.when(j >= NBUF)
            def _():
                pltpu.make_async_copy(ones_v, deg_sh.at[dst_v.at[j - NBUF]],
                                      dsem.at[b]).wait()

            pltpu.async_copy(ones_v, deg_sh.at[dst_v.at[j]], dsem.at[b],
                             add=True)
            return c

        lax.fori_loop(0, A_CHUNKS, hist_step, 0)
        for b in range(NBUF):
            pltpu.make_async_copy(
                ones_v, deg_sh.at[dst_v.at[A_CHUNKS - NBUF + b]],
                dsem.at[b]).wait()
        plsc.subcore_barrier()
        pltpu.sync_copy(deg_sh.at[pl.ds(base, ROWS_PER_TILE)], buf_v)
        for i in range(ROWS_PER_TILE // 16):
            sl = pl.ds(i * 16, 16)
            buf_v[sl] = _rsqrt16(buf_v[sl])
        pltpu.sync_copy(buf_v, dinv_hbm.at[pl.ds(base, ROWS_PER_TILE)])


def _agg_body(hsp_hbm, src_hbm, dst_hbm, out0_hbm, out1_hbm,
              src_r, dst_r, pk_v, f32_v, acc_sh, isem_s, isem_d, gsem, ssem):
    cid = lax.axis_index("c")
    sid = lax.axis_index("s")
    wid = cid * NS + sid
    zero16 = jnp.zeros((16,), jnp.float32)
    mask16 = jnp.full((16,), -65536, jnp.int32)

    def zrow(i, c):
        for t in range(D // 16):
            f32_v[0, i, pl.ds(t * 16, 16)] = zero16
        return c

    lax.fori_loop(0, CH, zrow, 0)
    base = sid * ROWS_PER_TILE
    for k in range(ROWS_PER_TILE // CH):
        pltpu.sync_copy(f32_v.at[0], acc_sh.at[pl.ds(base + k * CH, CH)])
    plsc.subcore_barrier()

    def fire_idx(gb, b):
        pltpu.async_copy(src_hbm.at[gb], src_r.at[b], isem_s.at[b])
        pltpu.async_copy(dst_hbm.at[gb], dst_r.at[b], isem_d.at[b])

    def wait_idx(gb, b):
        pltpu.make_async_copy(src_hbm.at[gb], src_r.at[b],
                              isem_s.at[b]).wait()
        pltpu.make_async_copy(dst_hbm.at[gb], dst_r.at[b],
                              isem_d.at[b]).wait()

    def convert(k, buf):
        def rows2(p, c):
            for rr in range(2):
                r = 2 * p + rr
                for kk in range(DW // 16):
                    w = pk_v[k, r, pl.ds(kk * 16, 16)]
                    lo = lax.bitcast_convert_type(
                        lax.shift_left(w, 16), jnp.float32)
                    hi = lax.bitcast_convert_type(
                        lax.bitwise_and(w, mask16), jnp.float32)
                    f32_v[buf, r, pl.ds(kk * 16, 16)] = lo
                    f32_v[buf, r, pl.ds(DW + kk * 16, 16)] = hi
            return c

        lax.fori_loop(0, CH // 2, rows2, 0)

    gbase = wid * GRP
    fire_idx(gbase, 0)

    def group(g, c):
        sb = lax.rem(g, 2)
        nb = lax.rem(g + 1, 2)
        gb = gbase + g
        wait_idx(gb, sb)

        @pl.when(g + 1 < GRP)
        def _():
            fire_idx(gb + 1, nb)

        gd = []
        for k in range(4):
            idx = src_r.at[sb].at[k // 2].at[pl.ds((k % 2) * CH, CH)]
            gd.append(pltpu.async_copy(hsp_hbm.at[idx], pk_v.at[k],
                                       gsem.at[k]))
        sd = {}
        for k in range(4):
            gd[k].wait()
            if k >= 2:
                sd[k - 2].wait()
            convert(k, k % 2)
            sd[k] = pltpu.async_copy(f32_v.at[k % 2],
                                     acc_sh.at[dst_r.at[sb].at[k]],
                                     ssem.at[k % 2], add=True)
        sd[2].wait()
        sd[3].wait()
        return c

    lax.fori_loop(0, GRP, group, 0)
    plsc.subcore_barrier()
    for k in range(ROWS_PER_TILE // CH):
        sl = pl.ds(base + k * CH, CH)
        pltpu.sync_copy(acc_sh.at[sl], f32_v.at[0])

        @pl.when(cid == 0)
        def _():
            pltpu.sync_copy(f32_v.at[0], out0_hbm.at[sl])

        @pl.when(cid == 1)
        def _():
            pltpu.sync_copy(f32_v.at[0], out1_hbm.at[sl])


def _mm_body(x_ref, w_ref, wp_ref, dinv_ref, o_ref, ob_ref):
    o_ref[...] = jnp.dot(x_ref[...], w_ref[...],
                         preferred_element_type=jnp.float32) * dinv_ref[...]
    ob_ref[...] = (jnp.dot(x_ref[...], wp_ref[...],
                           preferred_element_type=jnp.float32)
                   * dinv_ref[...]).astype(jnp.bfloat16)


def _final_body(a0_ref, a1_ref, hs_ref, dinv_ref, b_ref, a_ref, o_ref):
    o = dinv_ref[...] * (a0_ref[...] + a1_ref[...] + hs_ref[...]) + b_ref[...]
    slope = a_ref[0, 0]
    o_ref[...] = jnp.maximum(o, 0.0) + slope * jnp.minimum(o, 0.0)


def kernel(x, edge_index, W, b, prelu_a):
    pad = jnp.full((EPAD - E,), N, dtype=jnp.int32)
    src = jnp.concatenate([edge_index[0], pad])
    dst = jnp.concatenate([edge_index[1], pad])
    dstA = dst.reshape(NS, A_CHUNKS, CHUNK)
    src4 = src.reshape(NC * NS * GRP, 2, CHUNK)
    dst4 = dst.reshape(NC * NS * GRP, 4, CH)

    mesh = plsc.VectorSubcoreMesh(core_axis_name="c", subcore_axis_name="s",
                                  num_cores=NC, num_subcores=NS)

    deg_call = pl.kernel(
        _deg_body,
        out_type=jax.ShapeDtypeStruct((NPAD,), jnp.float32),
        mesh=mesh,
        scratch_types=[
            pltpu.VMEM((A_CHUNKS, CHUNK), jnp.int32),
            pltpu.VMEM((CHUNK,), jnp.float32),
            pltpu.VMEM((ROWS_PER_TILE,), jnp.float32),
            pltpu.VMEM_SHARED((NPAD,), jnp.float32),
            pltpu.SemaphoreType.DMA((NBUF,)),
        ],
    )
    dinv_pad = deg_call(dstA)
    dinv2d = dinv_pad.reshape(NPAD, 1)

    cols = jnp.stack([jnp.arange(DW, dtype=jnp.int32),
                      jnp.arange(DW, dtype=jnp.int32) + DW],
                     axis=1).reshape(D)
    nblk = 10
    rows = N // nblk
    hs, hsb = pl.pallas_call(
        _mm_body,
        grid=(nblk,),
        in_specs=[
            pl.BlockSpec((rows, D), lambda i: (i, 0)),
            pl.BlockSpec((D, D), lambda i: (0, 0)),
            pl.BlockSpec((D, D), lambda i: (0, 0)),
            pl.BlockSpec((rows, 1), lambda i: (i, 0)),
        ],
        out_specs=(pl.BlockSpec((rows, D), lambda i: (i, 0)),
                   pl.BlockSpec((rows, D), lambda i: (i, 0))),
        out_shape=(jax.ShapeDtypeStruct((NPAD, D), jnp.float32),
                   jax.ShapeDtypeStruct((NPAD, D), jnp.bfloat16)),
    )(x, W, W[:, cols], dinv2d)

    agg_call = pl.kernel(
        _agg_body,
        out_type=(jax.ShapeDtypeStruct((NPAD, D), jnp.float32),
                  jax.ShapeDtypeStruct((NPAD, D), jnp.float32)),
        mesh=mesh,
        compiler_params=pltpu.CompilerParams(use_tc_tiling_on_sc=False),
        scratch_types=[
            pltpu.VMEM((NBUF, 2, CHUNK), jnp.int32),
            pltpu.VMEM((NBUF, 4, CH), jnp.int32),
            pltpu.VMEM((4, CH, DW), jnp.int32),
            pltpu.VMEM((2, CH, D), jnp.float32),
            pltpu.VMEM_SHARED((NPAD, D), jnp.float32),
            pltpu.SemaphoreType.DMA((NBUF,)),
            pltpu.SemaphoreType.DMA((NBUF,)),
            pltpu.SemaphoreType.DMA((4,)),
            pltpu.SemaphoreType.DMA((2,)),
        ],
    )
    hs_pk = lax.bitcast_convert_type(hsb.reshape(NPAD, DW, 2), jnp.int32)
    agg0, agg1 = agg_call(hs_pk, src4, dst4)

    out = pl.pallas_call(
        _final_body,
        grid=(nblk,),
        in_specs=[
            pl.BlockSpec((rows, D), lambda i: (i, 0)),
            pl.BlockSpec((rows, D), lambda i: (i, 0)),
            pl.BlockSpec((rows, D), lambda i: (i, 0)),
            pl.BlockSpec((rows, 1), lambda i: (i, 0)),
            pl.BlockSpec((1, D), lambda i: (0, 0)),
            pl.BlockSpec(memory_space=pltpu.SMEM),
        ],
        out_specs=pl.BlockSpec((rows, D), lambda i: (i, 0)),
        out_shape=jax.ShapeDtypeStruct((N, D), jnp.float32),
    )(agg0, agg1, hs, dinv2d, b.reshape(1, D), prelu_a.reshape(1, 1))
    return out


# speedup vs baseline: 1.3378x; 1.3378x over previous
import jax
import jax.numpy as jnp
from jax import lax
from jax.experimental import pallas as pl
from jax.experimental.pallas import tpu as pltpu
from jax.experimental.pallas import tpu_sc as plsc

N = 10000
E = 320000
D = 128
NC = 2
NS = 16
NPAD = 10240
CHUNK = 128
EPAD = 327680
EPT_A = EPAD // NS
EPT_C = EPAD // (NC * NS)
A_CHUNKS = EPT_A // CHUNK
C_CHUNKS = EPT_C // CHUNK
ROWS_PER_TILE = NPAD // NS
WB = ROWS_PER_TILE // CHUNK
NBUF = 2
GRP = C_CHUNKS // 2
CH = 64
DW = D // 2


def _rsqrt16(d):
    i = lax.bitcast_convert_type(d, jnp.int32)
    magic = jnp.full((16,), 0x5F3759DF, jnp.int32)
    y = lax.bitcast_convert_type(magic - lax.shift_right_logical(i, 1),
                                 jnp.float32)
    for _ in range(3):
        y = y * (1.5 - 0.5 * d * y * y)
    return y


def _deg_body(dst_hbm, dinv_hbm, dst_v, ones_v, buf_v, deg_sh, dsem):
    cid = lax.axis_index("c")
    sid = lax.axis_index("s")

    @pl.when(cid == 0)
    def _():
        one16 = jnp.ones((16,), jnp.float32)
        for i in range(CHUNK // 16):
            ones_v[pl.ds(i * 16, 16)] = one16
        for i in range(ROWS_PER_TILE // 16):
            buf_v[pl.ds(i * 16, 16)] = one16
        base = sid * ROWS_PER_TILE
        pltpu.sync_copy(buf_v, deg_sh.at[pl.ds(base, ROWS_PER_TILE)])
        plsc.subcore_barrier()
        pltpu.sync_copy(dst_hbm.at[sid], dst_v)

        def hist_step(j, c):
            b = lax.rem(j, NBUF)

            @pl.when(j >= NBUF)
            def _():
                pltpu.make_async_copy(ones_v, deg_sh.at[dst_v.at[j - NBUF]],
                                      dsem.at[b]).wait()

            pltpu.async_copy(ones_v, deg_sh.at[dst_v.at[j]], dsem.at[b],
                             add=True)
            return c

        lax.fori_loop(0, A_CHUNKS, hist_step, 0)
        for b in range(NBUF):
            pltpu.make_async_copy(
                ones_v, deg_sh.at[dst_v.at[A_CHUNKS - NBUF + b]],
                dsem.at[b]).wait()
        plsc.subcore_barrier()
        pltpu.sync_copy(deg_sh.at[pl.ds(base, ROWS_PER_TILE)], buf_v)
        for i in range(ROWS_PER_TILE // 16):
            sl = pl.ds(i * 16, 16)
            buf_v[sl] = _rsqrt16(buf_v[sl])
        pltpu.sync_copy(buf_v, dinv_hbm.at[pl.ds(base, ROWS_PER_TILE)])


def _agg_body(hsp_hbm, src_hbm, dst_hbm, out0_hbm, out1_hbm,
              src_r, dst_r, pk_v, f32_v, acc_sh, isem_s, isem_d, gsem, ssem):
    cid = lax.axis_index("c")
    sid = lax.axis_index("s")
    wid = cid * NS + sid
    zero16 = jnp.zeros((16,), jnp.float32)
    mask16 = jnp.full((16,), -65536, jnp.int32)

    def zrow(i, c):
        for t in range(D // 16):
            f32_v[0, i, pl.ds(t * 16, 16)] = zero16
        return c

    lax.fori_loop(0, CH, zrow, 0)
    base = sid * ROWS_PER_TILE
    for k in range(ROWS_PER_TILE // CH):
        pltpu.sync_copy(f32_v.at[0], acc_sh.at[pl.ds(base + k * CH, CH)])
    plsc.subcore_barrier()

    def fire_idx(gb, b):
        pltpu.async_copy(src_hbm.at[gb], src_r.at[b], isem_s.at[b])
        pltpu.async_copy(dst_hbm.at[gb], dst_r.at[b], isem_d.at[b])

    def wait_idx(gb, b):
        pltpu.make_async_copy(src_hbm.at[gb], src_r.at[b],
                              isem_s.at[b]).wait()
        pltpu.make_async_copy(dst_hbm.at[gb], dst_r.at[b],
                              isem_d.at[b]).wait()

    def convert(k, buf):
        @plsc.parallel_loop(0, CH, step=1, unroll=4)
        def _(r):
            for kk in range(DW // 16):
                w = pk_v[k, r, pl.ds(kk * 16, 16)]
                lo = lax.bitcast_convert_type(
                    lax.shift_left(w, 16), jnp.float32)
                hi = lax.bitcast_convert_type(
                    lax.bitwise_and(w, mask16), jnp.float32)
                f32_v[buf, r, pl.ds(kk * 16, 16)] = lo
                f32_v[buf, r, pl.ds(DW + kk * 16, 16)] = hi

    gbase = wid * GRP
    fire_idx(gbase, 0)

    def group(g, c):
        sb = lax.rem(g, 2)
        nb = lax.rem(g + 1, 2)
        gb = gbase + g
        wait_idx(gb, sb)

        @pl.when(g + 1 < GRP)
        def _():
            fire_idx(gb + 1, nb)

        gd = []
        for k in range(4):
            idx = src_r.at[sb].at[k // 2].at[pl.ds((k % 2) * CH, CH)]
            gd.append(pltpu.async_copy(hsp_hbm.at[idx], pk_v.at[k],
                                       gsem.at[k]))
        sd = {}
        for k in range(4):
            gd[k].wait()
            if k >= 2:
                sd[k - 2].wait()
            convert(k, k % 2)
            sd[k] = pltpu.async_copy(f32_v.at[k % 2],
                                     acc_sh.at[dst_r.at[sb].at[k]],
                                     ssem.at[k % 2], add=True)
        sd[2].wait()
        sd[3].wait()
        return c

    lax.fori_loop(0, GRP, group, 0)
    plsc.subcore_barrier()
    for k in range(ROWS_PER_TILE // CH):
        sl = pl.ds(base + k * CH, CH)
        pltpu.sync_copy(acc_sh.at[sl], f32_v.at[0])

        @pl.when(cid == 0)
        def _():
            pltpu.sync_copy(f32_v.at[0], out0_hbm.at[sl])

        @pl.when(cid == 1)
        def _():
            pltpu.sync_copy(f32_v.at[0], out1_hbm.at[sl])


def _mm_body(x_ref, w_ref, wp_ref, dinv_ref, o_ref, ob_ref):
    o_ref[...] = jnp.dot(x_ref[...], w_ref[...],
                         preferred_element_type=jnp.float32) * dinv_ref[...]
    ob_ref[...] = (jnp.dot(x_ref[...], wp_ref[...],
                           preferred_element_type=jnp.float32)
                   * dinv_ref[...]).astype(jnp.bfloat16)


def _final_body(a0_ref, a1_ref, hs_ref, dinv_ref, b_ref, a_ref, o_ref):
    o = dinv_ref[...] * (a0_ref[...] + a1_ref[...] + hs_ref[...]) + b_ref[...]
    slope = a_ref[0, 0]
    o_ref[...] = jnp.maximum(o, 0.0) + slope * jnp.minimum(o, 0.0)


def kernel(x, edge_index, W, b, prelu_a):
    pad = jnp.full((EPAD - E,), N, dtype=jnp.int32)
    src = jnp.concatenate([edge_index[0], pad])
    dst = jnp.concatenate([edge_index[1], pad])
    dstA = dst.reshape(NS, A_CHUNKS, CHUNK)
    src4 = src.reshape(NC * NS * GRP, 2, CHUNK)
    dst4 = dst.reshape(NC * NS * GRP, 4, CH)

    mesh = plsc.VectorSubcoreMesh(core_axis_name="c", subcore_axis_name="s",
                                  num_cores=NC, num_subcores=NS)

    deg_call = pl.kernel(
        _deg_body,
        out_type=jax.ShapeDtypeStruct((NPAD,), jnp.float32),
        mesh=mesh,
        scratch_types=[
            pltpu.VMEM((A_CHUNKS, CHUNK), jnp.int32),
            pltpu.VMEM((CHUNK,), jnp.float32),
            pltpu.VMEM((ROWS_PER_TILE,), jnp.float32),
            pltpu.VMEM_SHARED((NPAD,), jnp.float32),
            pltpu.SemaphoreType.DMA((NBUF,)),
        ],
    )
    dinv_pad = deg_call(dstA)
    dinv2d = dinv_pad.reshape(NPAD, 1)

    cols = jnp.stack([jnp.arange(DW, dtype=jnp.int32),
                      jnp.arange(DW, dtype=jnp.int32) + DW],
                     axis=1).reshape(D)
    nblk = 10
    rows = N // nblk
    hs, hsb = pl.pallas_call(
        _mm_body,
        grid=(nblk,),
        in_specs=[
            pl.BlockSpec((rows, D), lambda i: (i, 0)),
            pl.BlockSpec((D, D), lambda i: (0, 0)),
            pl.BlockSpec((D, D), lambda i: (0, 0)),
            pl.BlockSpec((rows, 1), lambda i: (i, 0)),
        ],
        out_specs=(pl.BlockSpec((rows, D), lambda i: (i, 0)),
                   pl.BlockSpec((rows, D), lambda i: (i, 0))),
        out_shape=(jax.ShapeDtypeStruct((NPAD, D), jnp.float32),
                   jax.ShapeDtypeStruct((NPAD, D), jnp.bfloat16)),
    )(x, W, W[:, cols], dinv2d)

    agg_call = pl.kernel(
        _agg_body,
        out_type=(jax.ShapeDtypeStruct((NPAD, D), jnp.float32),
                  jax.ShapeDtypeStruct((NPAD, D), jnp.float32)),
        mesh=mesh,
        compiler_params=pltpu.CompilerParams(use_tc_tiling_on_sc=False),
        scratch_types=[
            pltpu.VMEM((NBUF, 2, CHUNK), jnp.int32),
            pltpu.VMEM((NBUF, 4, CH), jnp.int32),
            pltpu.VMEM((4, CH, DW), jnp.int32),
            pltpu.VMEM((2, CH, D), jnp.float32),
            pltpu.VMEM_SHARED((NPAD, D), jnp.float32),
            pltpu.SemaphoreType.DMA((NBUF,)),
            pltpu.SemaphoreType.DMA((NBUF,)),
            pltpu.SemaphoreType.DMA((4,)),
            pltpu.SemaphoreType.DMA((2,)),
        ],
    )
    hs_pk = lax.bitcast_convert_type(hsb.reshape(NPAD, DW, 2), jnp.int32)
    agg0, agg1 = agg_call(hs_pk, src4, dst4)

    out = pl.pallas_call(
        _final_body,
        grid=(nblk,),
        in_specs=[
            pl.BlockSpec((rows, D), lambda i: (i, 0)),
            pl.BlockSpec((rows, D), lambda i: (i, 0)),
            pl.BlockSpec((rows, D), lambda i: (i, 0)),
            pl.BlockSpec((rows, 1), lambda i: (i, 0)),
            pl.BlockSpec((1, D), lambda i: (0, 0)),
            pl.BlockSpec(memory_space=pltpu.SMEM),
        ],
        out_specs=pl.BlockSpec((rows, D), lambda i: (i, 0)),
        out_shape=jax.ShapeDtypeStruct((N, D), jnp.float32),
    )(agg0, agg1, hs, dinv2d, b.reshape(1, D), prelu_a.reshape(1, 1))
    return out


# speedup vs baseline: 2.2958x; 1.7161x over previous
import jax
import jax.numpy as jnp
from jax import lax
from jax.experimental import pallas as pl
from jax.experimental.pallas import tpu as pltpu
from jax.experimental.pallas import tpu_sc as plsc

N = 10000
E = 320000
D = 128
NC = 2
NS = 16
NPAD = 10240
CHUNK = 128
EPAD = 327680
EPT_A = EPAD // NS
EPT_C = EPAD // (NC * NS)
A_CHUNKS = EPT_A // CHUNK
C_CHUNKS = EPT_C // CHUNK
ROWS_PER_TILE = NPAD // NS
WB = ROWS_PER_TILE // CHUNK
NBUF = 2
GRP = C_CHUNKS // 2
CH = 64
DW = D // 2


def _rsqrt16(d):
    i = lax.bitcast_convert_type(d, jnp.int32)
    magic = jnp.full((16,), 0x5F3759DF, jnp.int32)
    y = lax.bitcast_convert_type(magic - lax.shift_right_logical(i, 1),
                                 jnp.float32)
    for _ in range(3):
        y = y * (1.5 - 0.5 * d * y * y)
    return y


def _deg_body(dst_hbm, dinv_hbm, dst_v, ones_v, buf_v, deg_sh, dsem):
    cid = lax.axis_index("c")
    sid = lax.axis_index("s")

    @pl.when(cid == 0)
    def _():
        one16 = jnp.ones((16,), jnp.float32)
        for i in range(CHUNK // 16):
            ones_v[pl.ds(i * 16, 16)] = one16
        for i in range(ROWS_PER_TILE // 16):
            buf_v[pl.ds(i * 16, 16)] = one16
        base = sid * ROWS_PER_TILE
        pltpu.sync_copy(buf_v, deg_sh.at[pl.ds(base, ROWS_PER_TILE)])
        plsc.subcore_barrier()
        pltpu.sync_copy(dst_hbm.at[sid], dst_v)

        def hist_step(j, c):
            b = lax.rem(j, NBUF)

            @pl.when(j >= NBUF)
            def _():
                pltpu.make_async_copy(ones_v, deg_sh.at[dst_v.at[j - NBUF]],
                                      dsem.at[b]).wait()

            pltpu.async_copy(ones_v, deg_sh.at[dst_v.at[j]], dsem.at[b],
                             add=True)
            return c

        lax.fori_loop(0, A_CHUNKS, hist_step, 0)
        for b in range(NBUF):
            pltpu.make_async_copy(
                ones_v, deg_sh.at[dst_v.at[A_CHUNKS - NBUF + b]],
                dsem.at[b]).wait()
        plsc.subcore_barrier()
        pltpu.sync_copy(deg_sh.at[pl.ds(base, ROWS_PER_TILE)], buf_v)
        for i in range(ROWS_PER_TILE // 16):
            sl = pl.ds(i * 16, 16)
            buf_v[sl] = _rsqrt16(buf_v[sl])
        pltpu.sync_copy(buf_v, dinv_hbm.at[pl.ds(base, ROWS_PER_TILE)])


def _agg_body(hsp_hbm, src_hbm, dst_hbm, out0_hbm, out1_hbm,
              src_r, dst_r, pk_v, f32_v, acc_sh, isem_s, isem_d, gsem, ssem):
    cid = lax.axis_index("c")
    sid = lax.axis_index("s")
    wid = cid * NS + sid
    zero16 = jnp.zeros((16,), jnp.float32)
    mask16 = jnp.full((16,), -65536, jnp.int32)

    def zrow(i, c):
        for t in range(D // 16):
            f32_v[0, i, pl.ds(t * 16, 16)] = zero16
        return c

    lax.fori_loop(0, CH, zrow, 0)
    base = sid * ROWS_PER_TILE
    for k in range(ROWS_PER_TILE // CH):
        pltpu.sync_copy(f32_v.at[0], acc_sh.at[pl.ds(base + k * CH, CH)])
    plsc.subcore_barrier()

    def fire_idx(gb, b):
        pltpu.async_copy(src_hbm.at[gb], src_r.at[b], isem_s.at[b])
        pltpu.async_copy(dst_hbm.at[gb], dst_r.at[b], isem_d.at[b])

    def wait_idx(gb, b):
        pltpu.make_async_copy(src_hbm.at[gb], src_r.at[b],
                              isem_s.at[b]).wait()
        pltpu.make_async_copy(dst_hbm.at[gb], dst_r.at[b],
                              isem_d.at[b]).wait()

    def convert(k, buf):
        @plsc.parallel_loop(0, CH, step=1, unroll=4)
        def _(r):
            for kk in range(DW // 16):
                w = pk_v[k, r, pl.ds(kk * 16, 16)]
                lo = lax.bitcast_convert_type(
                    lax.shift_left(w, 16), jnp.float32)
                hi = lax.bitcast_convert_type(
                    lax.bitwise_and(w, mask16), jnp.float32)
                f32_v[buf, r, pl.ds(kk * 16, 16)] = lo
                f32_v[buf, r, pl.ds(DW + kk * 16, 16)] = hi

    gbase = wid * GRP
    fire_idx(gbase, 0)

    def group(g, c):
        sb = lax.rem(g, 2)
        nb = lax.rem(g + 1, 2)
        gb = gbase + g
        wait_idx(gb, sb)

        @pl.when(g + 1 < GRP)
        def _():
            fire_idx(gb + 1, nb)

        gd = []
        for k in range(4):
            idx = src_r.at[sb].at[k // 2].at[pl.ds((k % 2) * CH, CH)]
            gd.append(pltpu.async_copy(hsp_hbm.at[idx], pk_v.at[k],
                                       gsem.at[k]))
        sd = {}
        for k in range(4):
            gd[k].wait()
            if k >= 2:
                sd[k - 2].wait()
            convert(k, k % 2)
            sd[k] = pltpu.async_copy(f32_v.at[k % 2],
                                     acc_sh.at[dst_r.at[sb].at[k]],
                                     ssem.at[k % 2], add=True)
        sd[2].wait()
        sd[3].wait()
        return c

    lax.fori_loop(0, GRP, group, 0)
    plsc.subcore_barrier()
    for k in range(ROWS_PER_TILE // CH):
        sl = pl.ds(base + k * CH, CH)
        pltpu.sync_copy(acc_sh.at[sl], f32_v.at[0])

        @pl.when(cid == 0)
        def _():
            pltpu.sync_copy(f32_v.at[0], out0_hbm.at[sl])

        @pl.when(cid == 1)
        def _():
            pltpu.sync_copy(f32_v.at[0], out1_hbm.at[sl])


def _mm_body(x_ref, w_ref, wp_ref, dinv_ref, o_ref, ob_ref):
    o_ref[...] = jnp.dot(x_ref[...], w_ref[...],
                         preferred_element_type=jnp.float32) * dinv_ref[...]
    ob_ref[...] = (jnp.dot(x_ref[...], wp_ref[...],
                           preferred_element_type=jnp.float32)
                   * dinv_ref[...]).astype(jnp.bfloat16)


def _final_body(a0_ref, a1_ref, hs_ref, dinv_ref, b_ref, a_ref, o_ref):
    o = dinv_ref[...] * (a0_ref[...] + a1_ref[...] + hs_ref[...]) + b_ref[...]
    slope = a_ref[0, 0]
    o_ref[...] = jnp.maximum(o, 0.0) + slope * jnp.minimum(o, 0.0)


def kernel(x, edge_index, W, b, prelu_a):
    pad = N + (jnp.arange(EPAD - E, dtype=jnp.int32) % (NPAD - N))
    src = jnp.concatenate([edge_index[0], pad])
    dst = jnp.concatenate([edge_index[1], pad])
    dstA = dst.reshape(NS, A_CHUNKS, CHUNK)
    src4 = src.reshape(NC * NS * GRP, 2, CHUNK)
    dst4 = dst.reshape(NC * NS * GRP, 4, CH)

    mesh = plsc.VectorSubcoreMesh(core_axis_name="c", subcore_axis_name="s",
                                  num_cores=NC, num_subcores=NS)

    deg_call = pl.kernel(
        _deg_body,
        out_type=jax.ShapeDtypeStruct((NPAD,), jnp.float32),
        mesh=mesh,
        scratch_types=[
            pltpu.VMEM((A_CHUNKS, CHUNK), jnp.int32),
            pltpu.VMEM((CHUNK,), jnp.float32),
            pltpu.VMEM((ROWS_PER_TILE,), jnp.float32),
            pltpu.VMEM_SHARED((NPAD,), jnp.float32),
            pltpu.SemaphoreType.DMA((NBUF,)),
        ],
    )
    dinv_pad = deg_call(dstA)
    dinv2d = dinv_pad.reshape(NPAD, 1)

    cols = jnp.stack([jnp.arange(DW, dtype=jnp.int32),
                      jnp.arange(DW, dtype=jnp.int32) + DW],
                     axis=1).reshape(D)
    nblk = 10
    rows = N // nblk
    hs, hsb = pl.pallas_call(
        _mm_body,
        grid=(nblk,),
        in_specs=[
            pl.BlockSpec((rows, D), lambda i: (i, 0)),
            pl.BlockSpec((D, D), lambda i: (0, 0)),
            pl.BlockSpec((D, D), lambda i: (0, 0)),
            pl.BlockSpec((rows, 1), lambda i: (i, 0)),
        ],
        out_specs=(pl.BlockSpec((rows, D), lambda i: (i, 0)),
                   pl.BlockSpec((rows, D), lambda i: (i, 0))),
        out_shape=(jax.ShapeDtypeStruct((NPAD, D), jnp.float32),
                   jax.ShapeDtypeStruct((NPAD, D), jnp.bfloat16)),
    )(x, W, W[:, cols], dinv2d)

    agg_call = pl.kernel(
        _agg_body,
        out_type=(jax.ShapeDtypeStruct((NPAD, D), jnp.float32),
                  jax.ShapeDtypeStruct((NPAD, D), jnp.float32)),
        mesh=mesh,
        compiler_params=pltpu.CompilerParams(use_tc_tiling_on_sc=False),
        scratch_types=[
            pltpu.VMEM((NBUF, 2, CHUNK), jnp.int32),
            pltpu.VMEM((NBUF, 4, CH), jnp.int32),
            pltpu.VMEM((4, CH, DW), jnp.int32),
            pltpu.VMEM((2, CH, D), jnp.float32),
            pltpu.VMEM_SHARED((NPAD, D), jnp.float32),
            pltpu.SemaphoreType.DMA((NBUF,)),
            pltpu.SemaphoreType.DMA((NBUF,)),
            pltpu.SemaphoreType.DMA((4,)),
            pltpu.SemaphoreType.DMA((2,)),
        ],
    )
    hs_pk = lax.bitcast_convert_type(hsb.reshape(NPAD, DW, 2), jnp.int32)
    agg0, agg1 = agg_call(hs_pk, src4, dst4)

    out = pl.pallas_call(
        _final_body,
        grid=(nblk,),
        in_specs=[
            pl.BlockSpec((rows, D), lambda i: (i, 0)),
            pl.BlockSpec((rows, D), lambda i: (i, 0)),
            pl.BlockSpec((rows, D), lambda i: (i, 0)),
            pl.BlockSpec((rows, 1), lambda i: (i, 0)),
            pl.BlockSpec((1, D), lambda i: (0, 0)),
            pl.BlockSpec(memory_space=pltpu.SMEM),
        ],
        out_specs=pl.BlockSpec((rows, D), lambda i: (i, 0)),
        out_shape=jax.ShapeDtypeStruct((N, D), jnp.float32),
    )(agg0, agg1, hs, dinv2d, b.reshape(1, D), prelu_a.reshape(1, 1))
    return out
